# Initial kernel scaffold; baseline (speedup 1.0000x reference)
import jax
import jax.numpy as jnp
from jax.experimental import pallas as pl


def kernel(location, region_id, coord_W, coord_b, region_table):
    raise NotImplementedError("write your pallas kernel here")


# speedup vs baseline: 2.0407x; 2.0407x over previous
import functools

import jax
import jax.numpy as jnp
from jax import lax
from jax.experimental import pallas as pl
from jax.experimental.pallas import tpu as pltpu
from jax.experimental.pallas import tpu_sc as plsc

_LAT_MIN, _LAT_MAX = -90.0, 90.0
_LON_MIN, _LON_MAX = -180.0, 180.0

_NC, _NS, _LANES = 2, 16, 16
_NW = _NC * _NS
_C = 2048
_IW = 128


def _sc_body(aug_hbm, idx_hbm, loc_hbm, coef_hbm, out_hbm,
             idx_v, out_v, loc_v, coef_v, sem):
    wid = lax.axis_index("s") * _NC + lax.axis_index("c")
    n = idx_hbm.shape[0] * _IW
    n_w = n // _NW
    lanes = lax.iota(jnp.int32, _LANES)

    pltpu.sync_copy(coef_hbm, coef_v)
    avs = [plsc.load_gather(coef_v, [jnp.full((_LANES,), o, jnp.int32)])
           for o in range(6)]
    cvs = [plsc.load_gather(coef_v, [jnp.full((_LANES,), 16 + o, jnp.int32)])
           for o in range(6)]

    def chunk_body(g, carry):
        base = pl.multiple_of(wid * n_w + g * _C, _C)
        row_base = pl.multiple_of(base // _IW, _C // _IW)
        pltpu.sync_copy(idx_hbm.at[pl.ds(row_base, _C // _IW)], idx_v)
        pltpu.sync_copy(loc_hbm.at[pl.ds(2 * base, 2 * _C)], loc_v)
        copies = [
            pltpu.async_copy(aug_hbm.at[idx_v.at[j]],
                             out_v.at[pl.ds(j * _IW, _IW)], sem)
            for j in range(_C // _IW)
        ]
        for cp in copies:
            cp.wait()

        def k_body(k, c):
            eidx = k * _LANES + lanes
            lat = plsc.load_gather(loc_v, [eidx * 2])
            lon = plsc.load_gather(loc_v, [eidx * 2 + 1])
            for o in range(6):
                col = jnp.full((_LANES,), o, jnp.int32)
                plsc.addupdate_scatter(out_v, [eidx, col],
                                       lat * avs[o] + lon * cvs[o])
            return c

        lax.fori_loop(0, _C // _LANES, k_body, 0, unroll=2)
        pltpu.sync_copy(out_v, out_hbm.at[pl.ds(base, _C)])
        return carry

    lax.fori_loop(0, n_w // _C, chunk_body, 0)


def kernel(location, region_id, coord_W, coord_b, region_table):
    B, L, _ = location.shape
    R = region_table.shape[0]
    n = B * L

    a = coord_W[:, 0] / (_LAT_MAX - _LAT_MIN)
    c = coord_W[:, 1] / (_LON_MAX - _LON_MIN)
    d = coord_b + 0.5 * coord_W[:, 0] + 0.5 * coord_W[:, 1]

    aug = jnp.concatenate(
        [jnp.broadcast_to(d, (R, 6)), region_table], axis=1)
    coefs = jnp.concatenate(
        [a, jnp.zeros((10,), jnp.float32), c, jnp.zeros((10,), jnp.float32)])

    idx_flat = region_id.reshape(n // _IW, _IW)
    loc_flat = location.reshape(n * 2)

    mesh = plsc.VectorSubcoreMesh(core_axis_name="c", subcore_axis_name="s")
    run = functools.partial(
        pl.kernel,
        mesh=mesh,
        out_type=jax.ShapeDtypeStruct((n, 16), jnp.float32),
        scratch_types=[
            pltpu.VMEM((_C // _IW, _IW), jnp.int32),
            pltpu.VMEM((_C, 16), jnp.float32),
            pltpu.VMEM((2 * _C,), jnp.float32),
            pltpu.VMEM((32,), jnp.float32),
            pltpu.SemaphoreType.DMA,
        ],
        compiler_params=pltpu.CompilerParams(
            needs_layout_passes=False, use_tc_tiling_on_sc=False),
    )(_sc_body)
    out = run(aug, idx_flat, loc_flat, coefs)
    return out.reshape(B, L, 16)


# speedup vs baseline: 15.9296x; 7.8060x over previous
import functools

import jax
import jax.numpy as jnp
from jax import lax
from jax.experimental import pallas as pl
from jax.experimental.pallas import tpu as pltpu
from jax.experimental.pallas import tpu_sc as plsc

_LAT_MIN, _LAT_MAX = -90.0, 90.0
_LON_MIN, _LON_MAX = -180.0, 180.0

_NC, _NS, _LANES = 2, 16, 16
_LB = 8
_W = 256
_NBUF = 2


def _splat(coef_v, i):
    return plsc.load_gather(coef_v, [jnp.full((_LANES,), i, jnp.int32)])


def _sc_body(idx_hbm, loc_hbm, tab_hbm, coef_hbm, out_hbm,
             idx_v, loc_v, out_v, plane_v, coef_v, sem_in, sem_out):
    cid = lax.axis_index("c")
    tid = lax.axis_index("s")
    l_total = idx_hbm.shape[0]
    b_half = idx_hbm.shape[1] // _NC
    b_base = cid * b_half
    n_lb = l_total // _LB
    n_bc = b_half // _W
    n_chunks = n_lb * n_bc

    is_region = tid >= 6

    pltpu.sync_copy(coef_hbm, coef_v)
    av = _splat(coef_v, tid)
    cv = _splat(coef_v, 16 + tid)
    dv = _splat(coef_v, 32 + tid)

    @pl.when(is_region)
    def _stage_plane():
        pltpu.make_async_copy(
            tab_hbm.at[tid - 6], plane_v, sem_in.at[0]).start()
        pltpu.make_async_copy(
            tab_hbm.at[tid - 6], plane_v, sem_in.at[0]).wait()

    def _in_start(c, p):
        lb = c // n_bc
        bc = c - lb * n_bc
        l0 = pl.multiple_of(lb * _LB, _LB)
        b0 = pl.multiple_of(b_base + bc * _W, _W)

        @pl.when(is_region)
        def _():
            pltpu.make_async_copy(
                idx_hbm.at[pl.ds(l0, _LB), pl.ds(b0, _W)],
                idx_v.at[p], sem_in.at[p]).start()

        @pl.when(jnp.logical_not(is_region))
        def _():
            pltpu.make_async_copy(
                loc_hbm.at[pl.ds(l0, _LB), :, pl.ds(b0, _W)],
                loc_v.at[p], sem_in.at[p]).start()

    def _in_wait(p):
        @pl.when(is_region)
        def _():
            pltpu.make_async_copy(
                idx_hbm.at[pl.ds(0, _LB), pl.ds(0, _W)],
                idx_v.at[p], sem_in.at[p]).wait()

        @pl.when(jnp.logical_not(is_region))
        def _():
            pltpu.make_async_copy(
                loc_hbm.at[pl.ds(0, _LB), :, pl.ds(0, _W)],
                loc_v.at[p], sem_in.at[p]).wait()

    def _out_start(c, p):
        lb = c // n_bc
        bc = c - lb * n_bc
        l0 = pl.multiple_of(lb * _LB, _LB)
        b0 = pl.multiple_of(b_base + bc * _W, _W)
        pltpu.make_async_copy(
            out_v.at[p],
            out_hbm.at[pl.ds(l0, _LB), tid, pl.ds(b0, _W)],
            sem_out.at[p]).start()

    def _out_wait(p):
        pltpu.make_async_copy(
            out_v.at[p],
            out_hbm.at[pl.ds(0, _LB), 0, pl.ds(0, _W)],
            sem_out.at[p]).wait()

    for p in range(_NBUF):
        _in_start(p, p)

    def chunk_body(c, carry):
        p = c % _NBUF
        _in_wait(p)

        @pl.when(c >= _NBUF)
        def _():
            _out_wait(p)

        @pl.when(is_region)
        def _compute_region():
            def body(i, acc):
                r = i // (_W // _LANES)
                k = i - r * (_W // _LANES)
                ids = idx_v[p, r, pl.ds(k * _LANES, _LANES)]
                out_v[p, r, pl.ds(k * _LANES, _LANES)] = (
                    plsc.load_gather(plane_v, [ids]))
                return acc
            lax.fori_loop(0, _LB * (_W // _LANES), body, 0, unroll=4)

        @pl.when(jnp.logical_not(is_region))
        def _compute_coord():
            def body(i, acc):
                r = i // (_W // _LANES)
                k = i - r * (_W // _LANES)
                lat = loc_v[p, r, 0, pl.ds(k * _LANES, _LANES)]
                lon = loc_v[p, r, 1, pl.ds(k * _LANES, _LANES)]
                out_v[p, r, pl.ds(k * _LANES, _LANES)] = (
                    lat * av + lon * cv + dv)
                return acc
            lax.fori_loop(0, _LB * (_W // _LANES), body, 0, unroll=4)

        _out_start(c, p)

        @pl.when(c + _NBUF < n_chunks)
        def _():
            _in_start(c + _NBUF, p)
        return carry

    lax.fori_loop(0, n_chunks, chunk_body, 0)

    for p in range(_NBUF):
        _out_wait(p)


def kernel(location, region_id, coord_W, coord_b, region_table):
    B, L, _ = location.shape
    R = region_table.shape[0]

    a = coord_W[:, 0] / (_LAT_MAX - _LAT_MIN)
    c = coord_W[:, 1] / (_LON_MAX - _LON_MIN)
    d = coord_b + 0.5 * coord_W[:, 0] + 0.5 * coord_W[:, 1]
    pad = jnp.zeros((10,), jnp.float32)
    coefs = jnp.concatenate([a, pad, c, pad, d, pad])

    idx_t = region_id.T
    loc_t = jnp.transpose(location, (1, 2, 0))
    tab_t = region_table.T

    mesh = plsc.VectorSubcoreMesh(core_axis_name="c", subcore_axis_name="s")
    run = functools.partial(
        pl.kernel,
        mesh=mesh,
        out_type=jax.ShapeDtypeStruct((L, 16, B), jnp.float32),
        scratch_types=[
            pltpu.VMEM((_NBUF, _LB, _W), jnp.int32),
            pltpu.VMEM((_NBUF, _LB, 2, _W), jnp.float32),
            pltpu.VMEM((_NBUF, _LB, _W), jnp.float32),
            pltpu.VMEM((R,), jnp.float32),
            pltpu.VMEM((48,), jnp.float32),
            pltpu.SemaphoreType.DMA((_NBUF,)),
            pltpu.SemaphoreType.DMA((_NBUF,)),
        ],
        compiler_params=pltpu.CompilerParams(
            needs_layout_passes=False, use_tc_tiling_on_sc=True),
    )(_sc_body)
    out_t = run(idx_t, loc_t, tab_t, coefs)
    return jnp.transpose(out_t, (2, 0, 1))
